# baseline (speedup 1.0000x reference)
import functools

import jax
import jax.numpy as jnp
from jax import lax
from jax.experimental import pallas as pl
from jax.experimental.pallas import tpu as pltpu
from jax.experimental.pallas import tpu_sc as plsc

D_INPUT = 128
MINUTE_SIZE = 4
HOUR_SIZE = 24
WEEKDAY_SIZE = 7
N_TIME = MINUTE_SIZE * HOUR_SIZE
N_FUSED = N_TIME * WEEKDAY_SIZE

_CHUNK = 320
_GROUPS = (128, 128, 64)


def _make_lookup(n_rows: int):
    info = plsc.get_sparse_core_info()
    nc, ns, nl = info.num_cores, info.num_subcores, info.num_lanes
    nw = nc * ns
    n_per_w = n_rows // nw
    assert n_rows % (nw * 2 * _CHUNK) == 0
    n_pairs = n_per_w // (2 * _CHUNK)
    rows_per_tile = N_FUSED // ns
    nseg = D_INPUT // nl

    mesh = plsc.VectorSubcoreMesh(core_axis_name="c", subcore_axis_name="s")

    @functools.partial(
        pl.kernel,
        mesh=mesh,
        out_type=jax.ShapeDtypeStruct((n_rows, D_INPUT), jnp.float32),
        scratch_types=[
            pltpu.VMEM((MINUTE_SIZE, D_INPUT), jnp.float32),
            pltpu.VMEM((HOUR_SIZE, D_INPUT), jnp.float32),
            pltpu.VMEM((WEEKDAY_SIZE, D_INPUT), jnp.float32),
            pltpu.VMEM((_CHUNK,), jnp.int32),
            pltpu.VMEM((_CHUNK,), jnp.int32),
            pltpu.VMEM((_CHUNK,), jnp.int32),
            pltpu.VMEM((_CHUNK,), jnp.int32),
            pltpu.VMEM((len(_GROUPS), 128), jnp.int32),
            pltpu.VMEM((len(_GROUPS), 128), jnp.int32),
            pltpu.VMEM((_CHUNK, D_INPUT), jnp.float32),
            pltpu.VMEM((_CHUNK, D_INPUT), jnp.float32),
            pltpu.VMEM_SHARED((N_FUSED, D_INPUT), jnp.float32),
            pltpu.SemaphoreType.DMA,
            pltpu.SemaphoreType.DMA,
            pltpu.SemaphoreType.DMA,
            pltpu.SemaphoreType.DMA,
            pltpu.SemaphoreType.DMA,
            pltpu.SemaphoreType.DMA,
        ],
    )
    def lookup(
        min_hbm, hr_hbm, wd_hbm, t_hbm, w_hbm, out_hbm,
        min_v, hr_v, wdt_v, t0, t1, w0, w1, idx0, idx1, rows0, rows1,
        shared_tbl,
        gsem0, gsem1, ssem0, ssem1, lsem0, lsem1,
    ):
        sid = lax.axis_index("s")
        wid = sid * nc + lax.axis_index("c")
        base_w = wid * n_per_w

        pltpu.sync_copy(min_hbm, min_v)
        pltpu.sync_copy(hr_hbm, hr_v)
        pltpu.sync_copy(wd_hbm, wdt_v)

        row0 = sid * rows_per_tile

        def build_row(k, carry):
            r = row0 + k
            t96 = r // WEEKDAY_SIZE
            wd = r % WEEKDAY_SIZE
            h = t96 // MINUTE_SIZE
            m = t96 % MINUTE_SIZE
            for seg in range(nseg):
                sl = pl.ds(seg * nl, nl)
                rows0[k, sl] = hr_v[h, sl] + min_v[m, sl] + wdt_v[wd, sl]
            return carry

        lax.fori_loop(0, rows_per_tile, build_row, 0)
        pltpu.sync_copy(
            rows0.at[pl.ds(0, rows_per_tile)],
            shared_tbl.at[pl.ds(row0, rows_per_tile)],
        )
        plsc.subcore_barrier()

        idx_bufs = (idx0, idx1)
        row_bufs = (rows0, rows1)
        gather_sems = (gsem0, gsem1)
        store_sems = (ssem0, ssem1)
        t_bufs = (t0, t1)
        w_bufs = (w0, w1)
        load_sems = (lsem0, lsem1)

        def _wait_gather_and_store(pb, pbase):
            pltpu.make_async_copy(
                out_hbm.at[pl.ds(0, _CHUNK)], row_bufs[pb], gather_sems[pb]
            ).wait()
            pltpu.async_copy(
                row_bufs[pb],
                out_hbm.at[pl.ds(pbase, _CHUNK)],
                store_sems[pb],
            )

        pltpu.async_copy(t_hbm.at[pl.ds(base_w, _CHUNK)], t0, lsem0)
        pltpu.async_copy(w_hbm.at[pl.ds(base_w, _CHUNK)], w0, lsem0)

        def body(gg, carry):
            for b in range(2):
                idxb, rowsb = idx_bufs[b], row_bufs[b]
                tb, wb, lsemb = t_bufs[b], w_bufs[b], load_sems[b]
                base = base_w + (gg * 2 + b) * _CHUNK

                pltpu.make_async_copy(
                    t_hbm.at[pl.ds(base, _CHUNK)], tb, lsemb
                ).wait()
                pltpu.make_async_copy(
                    w_hbm.at[pl.ds(base, _CHUNK)], wb, lsemb
                ).wait()

                nb = 1 - b
                nbase = base + _CHUNK

                def _prefetch(nb=nb, nbase=nbase):
                    pltpu.async_copy(
                        t_hbm.at[pl.ds(nbase, _CHUNK)], t_bufs[nb],
                        load_sems[nb],
                    )
                    pltpu.async_copy(
                        w_hbm.at[pl.ds(nbase, _CHUNK)], w_bufs[nb],
                        load_sems[nb],
                    )

                if b == 0:
                    _prefetch()
                else:
                    pl.when(gg + 1 < n_pairs)(_prefetch)

                for i in range(_CHUNK // nl):
                    t = tb[pl.ds(i * nl, nl)]
                    w = wb[pl.ds(i * nl, nl)]
                    idxb[i // (128 // nl), pl.ds((i % (128 // nl)) * nl, nl)] = (
                        t * WEEKDAY_SIZE + w
                    )

                @pl.when(gg > 0)
                def _drain(rowsb=rowsb, b=b, base=base):
                    pltpu.make_async_copy(
                        rowsb, out_hbm.at[pl.ds(base, _CHUNK)], store_sems[b]
                    ).wait()

                off = 0
                for j, gsz in enumerate(_GROUPS):
                    idx_view = (
                        idxb.at[j] if gsz == 128
                        else idxb.at[j, pl.ds(0, gsz)]
                    )
                    pltpu.async_copy(
                        shared_tbl.at[idx_view],
                        rowsb.at[pl.ds(off, gsz)],
                        gather_sems[b],
                    )
                    off += gsz

                if b == 1:
                    _wait_gather_and_store(0, base - _CHUNK)
                else:
                    pl.when(gg > 0)(
                        functools.partial(
                            _wait_gather_and_store, 1, base - _CHUNK
                        )
                    )
            return carry

        lax.fori_loop(0, n_pairs, body, 0)

        last_base = base_w + n_per_w - _CHUNK
        pltpu.make_async_copy(
            out_hbm.at[pl.ds(0, _CHUNK)], rows1, gsem1
        ).wait()
        pltpu.async_copy(rows1, out_hbm.at[pl.ds(last_base, _CHUNK)], ssem1)
        pltpu.make_async_copy(
            rows0, out_hbm.at[pl.ds(base_w, _CHUNK)], ssem0
        ).wait()
        pltpu.make_async_copy(
            rows1, out_hbm.at[pl.ds(base_w, _CHUNK)], ssem1
        ).wait()

    return lookup


def kernel(time, weekday, minute_embed, hour_embed, weekday_embed):
    b, t = time.shape
    n = b * t
    t_flat = time.reshape(n).astype(jnp.int32)
    w_flat = weekday.reshape(n).astype(jnp.int32)
    out = _make_lookup(n)(
        minute_embed, hour_embed, weekday_embed, t_flat, w_flat
    )
    return out.reshape(b, t, D_INPUT)


# speedup vs baseline: 1.0091x; 1.0091x over previous
import functools

import jax
import jax.numpy as jnp
from jax import lax
from jax.experimental import pallas as pl
from jax.experimental.pallas import tpu as pltpu
from jax.experimental.pallas import tpu_sc as plsc

D_INPUT = 128
MINUTE_SIZE = 4
HOUR_SIZE = 24
WEEKDAY_SIZE = 7
N_TIME = MINUTE_SIZE * HOUR_SIZE
N_FUSED = N_TIME * WEEKDAY_SIZE

_CHUNK = 256


def _make_lookup(n_rows: int):
    info = plsc.get_sparse_core_info()
    nc, ns, nl = info.num_cores, info.num_subcores, info.num_lanes
    nw = nc * ns
    n_per_w = n_rows // nw
    assert n_rows % (nw * 2 * _CHUNK) == 0
    n_pairs = n_per_w // (2 * _CHUNK)
    rows_per_tile = N_FUSED // ns
    nseg = D_INPUT // nl

    mesh = plsc.VectorSubcoreMesh(core_axis_name="c", subcore_axis_name="s")

    @functools.partial(
        pl.kernel,
        mesh=mesh,
        out_type=jax.ShapeDtypeStruct((n_rows, D_INPUT), jnp.float32),
        scratch_types=[
            pltpu.VMEM((MINUTE_SIZE, D_INPUT), jnp.float32),
            pltpu.VMEM((HOUR_SIZE, D_INPUT), jnp.float32),
            pltpu.VMEM((WEEKDAY_SIZE, D_INPUT), jnp.float32),
            pltpu.VMEM((_CHUNK,), jnp.int32),
            pltpu.VMEM((_CHUNK,), jnp.int32),
            pltpu.VMEM((_CHUNK,), jnp.int32),
            pltpu.VMEM((_CHUNK,), jnp.int32),
            pltpu.VMEM((_CHUNK // 128, 128), jnp.int32),
            pltpu.VMEM((_CHUNK // 128, 128), jnp.int32),
            pltpu.VMEM((_CHUNK, D_INPUT), jnp.float32),
            pltpu.VMEM((_CHUNK, D_INPUT), jnp.float32),
            pltpu.VMEM_SHARED((N_FUSED, D_INPUT), jnp.float32),
            pltpu.SemaphoreType.DMA,
            pltpu.SemaphoreType.DMA,
            pltpu.SemaphoreType.DMA,
            pltpu.SemaphoreType.DMA,
            pltpu.SemaphoreType.DMA,
            pltpu.SemaphoreType.DMA,
        ],
    )
    def lookup(
        min_hbm, hr_hbm, wd_hbm, t_hbm, w_hbm, out_hbm,
        min_v, hr_v, wdt_v, t0, t1, w0, w1, idx0, idx1, rows0, rows1,
        shared_tbl,
        gsem0, gsem1, ssem0, ssem1, lsem0, lsem1,
    ):
        sid = lax.axis_index("s")
        wid = sid * nc + lax.axis_index("c")
        base_w = wid * n_per_w

        pltpu.sync_copy(min_hbm, min_v)
        pltpu.sync_copy(hr_hbm, hr_v)
        pltpu.sync_copy(wd_hbm, wdt_v)

        row0 = sid * rows_per_tile

        def build_row(k, carry):
            r = row0 + k
            t96 = r // WEEKDAY_SIZE
            wd = r % WEEKDAY_SIZE
            h = t96 // MINUTE_SIZE
            m = t96 % MINUTE_SIZE
            for seg in range(nseg):
                sl = pl.ds(seg * nl, nl)
                rows0[k, sl] = hr_v[h, sl] + min_v[m, sl] + wdt_v[wd, sl]
            return carry

        lax.fori_loop(0, rows_per_tile, build_row, 0)
        pltpu.sync_copy(
            rows0.at[pl.ds(0, rows_per_tile)],
            shared_tbl.at[pl.ds(row0, rows_per_tile)],
        )
        plsc.subcore_barrier()

        idx_bufs = (idx0, idx1)
        row_bufs = (rows0, rows1)
        gather_sems = (gsem0, gsem1)
        store_sems = (ssem0, ssem1)
        t_bufs = (t0, t1)
        w_bufs = (w0, w1)
        load_sems = (lsem0, lsem1)

        def _wait_gather_and_store(pb, pbase):
            pltpu.make_async_copy(
                out_hbm.at[pl.ds(0, _CHUNK)], row_bufs[pb], gather_sems[pb]
            ).wait()
            pltpu.async_copy(
                row_bufs[pb],
                out_hbm.at[pl.ds(pbase, _CHUNK)],
                store_sems[pb],
            )

        pltpu.async_copy(t_hbm.at[pl.ds(base_w, _CHUNK)], t0, lsem0)
        pltpu.async_copy(w_hbm.at[pl.ds(base_w, _CHUNK)], w0, lsem0)

        def body(gg, carry):
            for b in range(2):
                idxb, rowsb = idx_bufs[b], row_bufs[b]
                tb, wb, lsemb = t_bufs[b], w_bufs[b], load_sems[b]
                base = base_w + (gg * 2 + b) * _CHUNK

                pltpu.make_async_copy(
                    t_hbm.at[pl.ds(base, _CHUNK)], tb, lsemb
                ).wait()
                pltpu.make_async_copy(
                    w_hbm.at[pl.ds(base, _CHUNK)], wb, lsemb
                ).wait()

                nb = 1 - b
                nbase = base + _CHUNK

                def _prefetch(nb=nb, nbase=nbase):
                    pltpu.async_copy(
                        t_hbm.at[pl.ds(nbase, _CHUNK)], t_bufs[nb],
                        load_sems[nb],
                    )
                    pltpu.async_copy(
                        w_hbm.at[pl.ds(nbase, _CHUNK)], w_bufs[nb],
                        load_sems[nb],
                    )

                if b == 0:
                    _prefetch()
                else:
                    pl.when(gg + 1 < n_pairs)(_prefetch)

                for i in range(_CHUNK // nl):
                    t = tb[pl.ds(i * nl, nl)]
                    w = wb[pl.ds(i * nl, nl)]
                    idxb[i // (128 // nl), pl.ds((i % (128 // nl)) * nl, nl)] = (
                        t * WEEKDAY_SIZE + w
                    )

                @pl.when(gg > 0)
                def _drain(rowsb=rowsb, b=b, base=base):
                    pltpu.make_async_copy(
                        rowsb, out_hbm.at[pl.ds(base, _CHUNK)], store_sems[b]
                    ).wait()

                for j in range(_CHUNK // 128):
                    pltpu.async_copy(
                        shared_tbl.at[idxb.at[j]],
                        rowsb.at[pl.ds(j * 128, 128)],
                        gather_sems[b],
                    )

                if b == 1:
                    _wait_gather_and_store(0, base - _CHUNK)
                else:
                    pl.when(gg > 0)(
                        functools.partial(
                            _wait_gather_and_store, 1, base - _CHUNK
                        )
                    )
            return carry

        lax.fori_loop(0, n_pairs, body, 0)

        last_base = base_w + n_per_w - _CHUNK
        pltpu.make_async_copy(
            out_hbm.at[pl.ds(0, _CHUNK)], rows1, gsem1
        ).wait()
        pltpu.async_copy(rows1, out_hbm.at[pl.ds(last_base, _CHUNK)], ssem1)
        pltpu.make_async_copy(
            rows0, out_hbm.at[pl.ds(base_w, _CHUNK)], ssem0
        ).wait()
        pltpu.make_async_copy(
            rows1, out_hbm.at[pl.ds(base_w, _CHUNK)], ssem1
        ).wait()

    return lookup


def kernel(time, weekday, minute_embed, hour_embed, weekday_embed):
    b, t = time.shape
    n = b * t
    t_flat = time.reshape(n).astype(jnp.int32)
    w_flat = weekday.reshape(n).astype(jnp.int32)
    out = _make_lookup(n)(
        minute_embed, hour_embed, weekday_embed, t_flat, w_flat
    )
    return out.reshape(b, t, D_INPUT)
